# Initial kernel scaffold; baseline (speedup 1.0000x reference)
import jax
import jax.numpy as jnp
from jax.experimental import pallas as pl


def kernel(x, edge_index, edge_attr, W1, b1, W2, b2, Ws, bs, Wn, bn):
    raise NotImplementedError("write your pallas kernel here")


# speedup vs baseline: 2.7313x; 2.7313x over previous
import functools

import jax
import jax.numpy as jnp
from jax import lax
from jax.experimental import pallas as pl
from jax.experimental.pallas import tpu as pltpu
from jax.experimental.pallas import tpu_sc as plsc

N, E, D, ED = 10000, 320000, 128, 17
NC, NS = 2, 16
NW = NC * NS
EPW = E // NW
CH = 80
NCHUNK = EPW // CH
NP = 10240
RPT = NP // NS
RCH = 128
NV = D // 16


def _mlp_body(ea, w1, b1, w2, b2, o):
    h = jnp.maximum(
        jnp.dot(ea[...], w1[...], preferred_element_type=jnp.float32) + b1[...],
        0.0)
    o[...] = jnp.dot(h, w2[...], preferred_element_type=jnp.float32) + b2[...]


def _edge_mlp(edge_attr, W1, b1, W2, b2):
    BE = 3200
    return pl.pallas_call(
        _mlp_body,
        grid=(E // BE,),
        in_specs=[
            pl.BlockSpec((BE, ED), lambda i: (i, 0)),
            pl.BlockSpec((ED, D), lambda i: (0, 0)),
            pl.BlockSpec((1, D), lambda i: (0, 0)),
            pl.BlockSpec((D, D), lambda i: (0, 0)),
            pl.BlockSpec((1, D), lambda i: (0, 0)),
        ],
        out_specs=pl.BlockSpec((BE, D), lambda i: (i, 0)),
        out_shape=jax.ShapeDtypeStruct((E, D), jnp.float32),
    )(edge_attr, W1, b1.reshape(1, D), W2, b2.reshape(1, D))


def _sc_body(src_hbm, dst_hbm, x_hbm, w_hbm, out_hbm, src_v, dst_v, xg_v, w_v,
             buf_v, agg_sh, sem):
    cid = lax.axis_index("c")
    sid = lax.axis_index("s")
    wid = sid * NC + cid

    def zrow(i, c):
        for j in range(NV):
            buf_v[i, pl.ds(j * 16, 16)] = jnp.zeros((16,), jnp.float32)
        return c
    lax.fori_loop(0, RCH, zrow, 0)

    def zcp(k, c):
        pltpu.sync_copy(buf_v, agg_sh.at[pl.ds(sid * RPT + k * RCH, RCH)])
        return c
    lax.fori_loop(0, RPT // RCH, zcp, 0)
    plsc.subcore_barrier()

    def chunk(g, c):
        base = wid * EPW + g * CH
        pltpu.sync_copy(src_hbm.at[pl.ds(base, CH)], src_v)
        pltpu.sync_copy(dst_hbm.at[pl.ds(base, CH)], dst_v)
        gather = pltpu.async_copy(x_hbm.at[src_v], xg_v, sem)
        pltpu.sync_copy(w_hbm.at[pl.ds(base, CH)], w_v)
        gather.wait()

        def mul_row(i, cc):
            for j in range(NV):
                sl = pl.ds(j * 16, 16)
                w_v[i, sl] = w_v[i, sl] * xg_v[i, sl]
            return cc
        lax.fori_loop(0, CH, mul_row, 0)
        pltpu.sync_copy(w_v, agg_sh.at[dst_v], add=True)
        return c
    lax.fori_loop(0, NCHUNK, chunk, 0)
    plsc.subcore_barrier()

    def outcp(k, c):
        r = sid * RPT + k * RCH
        pltpu.sync_copy(agg_sh.at[pl.ds(r, RCH)], buf_v)
        pltpu.sync_copy(buf_v, out_hbm.at[pl.ds(cid * NP + r, RCH)])
        return c
    lax.fori_loop(0, RPT // RCH, outcp, 0)


_sc_gather_scatter = functools.partial(
    pl.kernel,
    out_type=jax.ShapeDtypeStruct((NC * NP, D), jnp.float32),
    mesh=plsc.VectorSubcoreMesh(core_axis_name="c", subcore_axis_name="s"),
    scratch_types=[
        pltpu.VMEM((CH,), jnp.int32),
        pltpu.VMEM((CH,), jnp.int32),
        pltpu.VMEM((CH, D), jnp.float32),
        pltpu.VMEM((CH, D), jnp.float32),
        pltpu.VMEM((RCH, D), jnp.float32),
        pltpu.VMEM_SHARED((NP, D), jnp.float32),
        pltpu.SemaphoreType.DMA,
    ],
)(_sc_body)


def _out_body(x, p, ws, bs, wn, bn, o):
    agg = p[0] + p[1]
    o[...] = jnp.maximum(
        jnp.dot(x[...], ws[...], preferred_element_type=jnp.float32) + bs[...]
        + jnp.dot(agg, wn[...], preferred_element_type=jnp.float32) + bn[...],
        0.0)


def _out_lin(x, partials, Ws, bs, Wn, bn):
    BN = 2000
    return pl.pallas_call(
        _out_body,
        grid=(N // BN,),
        in_specs=[
            pl.BlockSpec((BN, D), lambda i: (i, 0)),
            pl.BlockSpec((NC, BN, D), lambda i: (0, i, 0)),
            pl.BlockSpec((D, D), lambda i: (0, 0)),
            pl.BlockSpec((1, D), lambda i: (0, 0)),
            pl.BlockSpec((D, D), lambda i: (0, 0)),
            pl.BlockSpec((1, D), lambda i: (0, 0)),
        ],
        out_specs=pl.BlockSpec((BN, D), lambda i: (i, 0)),
        out_shape=jax.ShapeDtypeStruct((N, D), jnp.float32),
    )(x, partials, Ws, bs.reshape(1, D), Wn, bn.reshape(1, D))


def kernel(x, edge_index, edge_attr, W1, b1, W2, b2, Ws, bs, Wn, bn):
    w = _edge_mlp(edge_attr, W1, b1, W2, b2)
    partials = _sc_gather_scatter(edge_index[0], edge_index[1], x, w)
    return _out_lin(x, partials.reshape(NC, NP, D)[:, :N], Ws, bs, Wn, bn)


# speedup vs baseline: 4.1315x; 1.5126x over previous
import functools

import jax
import jax.numpy as jnp
from jax import lax
from jax.experimental import pallas as pl
from jax.experimental.pallas import tpu as pltpu
from jax.experimental.pallas import tpu_sc as plsc

N, E, D, ED = 10000, 320000, 128, 17
NC, NS = 2, 16
NW = NC * NS
EPW = 10240
CH = 80
NP = 10240
RPT = NP // NS
RCH = 80
NV = D // 16


def _mlp_body(ea, w1, b1, w2, b2, o):
    h = jnp.maximum(
        jnp.dot(ea[...], w1[...], preferred_element_type=jnp.float32) + b1[...],
        0.0)
    o[...] = jnp.dot(h, w2[...], preferred_element_type=jnp.float32) + b2[...]


def _edge_mlp(edge_attr, W1, b1, W2, b2):
    BE = 3200
    return pl.pallas_call(
        _mlp_body,
        grid=(E // BE,),
        in_specs=[
            pl.BlockSpec((BE, ED), lambda i: (i, 0)),
            pl.BlockSpec((ED, D), lambda i: (0, 0)),
            pl.BlockSpec((1, D), lambda i: (0, 0)),
            pl.BlockSpec((D, D), lambda i: (0, 0)),
            pl.BlockSpec((1, D), lambda i: (0, 0)),
        ],
        out_specs=pl.BlockSpec((BE, D), lambda i: (i, 0)),
        out_shape=jax.ShapeDtypeStruct((E, D), jnp.float32),
    )(edge_attr, W1, b1.reshape(1, D), W2, b2.reshape(1, D))


def _sc_body(src_hbm, dst_hbm, x_hbm, w_hbm, out_hbm,
             s0, s1, d0, d1, d2, d3, xg0, xg1, wv0, wv1, agg_sh,
             si0, si1, sd0, sd1, sd2, sd3, sg0, sg1, sw0, sw1, ss0, ss1):
    cid = lax.axis_index("c")
    sid = lax.axis_index("s")
    wid = sid * NC + cid
    ebase = wid * EPW
    nch = jnp.minimum(EPW, E - ebase) // CH

    sbufs = (s0, s1)
    dbufs = (d0, d1, d2, d3)
    xgs = (xg0, xg1)
    wvs = (wv0, wv1)
    sis = (si0, si1)
    sds = (sd0, sd1, sd2, sd3)
    sgs = (sg0, sg1)
    sws = (sw0, sw1)
    sss = (ss0, ss1)

    def zrow(i, c):
        for j in range(NV):
            xg0[i, pl.ds(j * 16, 16)] = jnp.zeros((16,), jnp.float32)
        return c
    lax.fori_loop(0, RCH, zrow, 0)

    def zcp(k, c):
        pltpu.sync_copy(xg0, agg_sh.at[pl.ds(sid * RPT + k * RCH, RCH)])
        return c
    lax.fori_loop(0, RPT // RCH, zcp, 0)
    plsc.subcore_barrier()

    def do_chunk(g, j):
        p = j % 2
        q = 1 - p

        @pl.when(g + 1 < nch)
        def _():
            pltpu.make_async_copy(src_hbm.at[pl.ds(0, CH)], sbufs[q],
                                  sis[q]).wait()
            pltpu.make_async_copy(dst_hbm.at[pl.ds(0, CH)],
                                  dbufs[(j + 1) % 4], sds[(j + 1) % 4]).wait()

            @pl.when(g >= 1)
            def _():
                pltpu.make_async_copy(
                    wvs[q], agg_sh.at[dbufs[(j + 3) % 4]], sss[q]).wait()
            pltpu.async_copy(x_hbm.at[sbufs[q]], xgs[q], sgs[q])
            pltpu.async_copy(w_hbm.at[pl.ds(ebase + (g + 1) * CH, CH)],
                             wvs[q], sws[q])

        pltpu.make_async_copy(x_hbm.at[sbufs[p]], xgs[p], sgs[p]).wait()
        pltpu.make_async_copy(w_hbm.at[pl.ds(0, CH)], wvs[p], sws[p]).wait()

        def mul_row(i, cc):
            for v in range(NV):
                sl = pl.ds(v * 16, 16)
                wvs[p][i, sl] = wvs[p][i, sl] * xgs[p][i, sl]
            return cc
        lax.fori_loop(0, CH, mul_row, 0)

        pltpu.async_copy(wvs[p], agg_sh.at[dbufs[j % 4]], sss[p], add=True)

        @pl.when(g + 2 < nch)
        def _():
            b2 = ebase + (g + 2) * CH
            pltpu.async_copy(src_hbm.at[pl.ds(b2, CH)], sbufs[p], sis[p])
            pltpu.async_copy(dst_hbm.at[pl.ds(b2, CH)], dbufs[(j + 2) % 4],
                             sds[(j + 2) % 4])

    pltpu.sync_copy(src_hbm.at[pl.ds(ebase, CH)], s0)
    pltpu.sync_copy(dst_hbm.at[pl.ds(ebase, CH)], d0)
    pltpu.async_copy(x_hbm.at[s0], xg0, sg0)
    pltpu.async_copy(w_hbm.at[pl.ds(ebase, CH)], wv0, sw0)
    pltpu.async_copy(src_hbm.at[pl.ds(ebase + CH, CH)], s1, si1)
    pltpu.async_copy(dst_hbm.at[pl.ds(ebase + CH, CH)], d1, sd1)

    def quad(i, c):
        g = i * 4
        for j in range(4):
            do_chunk(g + j, j)
        return c
    lax.fori_loop(0, nch // 4, quad, 0)

    pltpu.make_async_copy(wvs[0], agg_sh.at[dbufs[2]], sss[0]).wait()
    pltpu.make_async_copy(wvs[1], agg_sh.at[dbufs[3]], sss[1]).wait()
    plsc.subcore_barrier()

    def outcp(k, c):
        r = sid * RPT + k * RCH
        pltpu.sync_copy(agg_sh.at[pl.ds(r, RCH)], xg0)
        pltpu.sync_copy(xg0, out_hbm.at[pl.ds(cid * NP + r, RCH)])
        return c
    lax.fori_loop(0, RPT // RCH, outcp, 0)


_sc_gather_scatter = functools.partial(
    pl.kernel,
    out_type=jax.ShapeDtypeStruct((NC * NP, D), jnp.float32),
    mesh=plsc.VectorSubcoreMesh(core_axis_name="c", subcore_axis_name="s"),
    scratch_types=[
        pltpu.VMEM((CH,), jnp.int32),
        pltpu.VMEM((CH,), jnp.int32),
        pltpu.VMEM((CH,), jnp.int32),
        pltpu.VMEM((CH,), jnp.int32),
        pltpu.VMEM((CH,), jnp.int32),
        pltpu.VMEM((CH,), jnp.int32),
        pltpu.VMEM((CH, D), jnp.float32),
        pltpu.VMEM((CH, D), jnp.float32),
        pltpu.VMEM((CH, D), jnp.float32),
        pltpu.VMEM((CH, D), jnp.float32),
        pltpu.VMEM_SHARED((NP, D), jnp.float32),
    ] + [pltpu.SemaphoreType.DMA] * 12,
)(_sc_body)


def _out_body(x, p, ws, bs, wn, bn, o):
    agg = p[0] + p[1]
    o[...] = jnp.maximum(
        jnp.dot(x[...], ws[...], preferred_element_type=jnp.float32) + bs[...]
        + jnp.dot(agg, wn[...], preferred_element_type=jnp.float32) + bn[...],
        0.0)


def _out_lin(x, partials, Ws, bs, Wn, bn):
    BN = 2000
    return pl.pallas_call(
        _out_body,
        grid=(N // BN,),
        in_specs=[
            pl.BlockSpec((BN, D), lambda i: (i, 0)),
            pl.BlockSpec((NC, BN, D), lambda i: (0, i, 0)),
            pl.BlockSpec((D, D), lambda i: (0, 0)),
            pl.BlockSpec((1, D), lambda i: (0, 0)),
            pl.BlockSpec((D, D), lambda i: (0, 0)),
            pl.BlockSpec((1, D), lambda i: (0, 0)),
        ],
        out_specs=pl.BlockSpec((BN, D), lambda i: (i, 0)),
        out_shape=jax.ShapeDtypeStruct((N, D), jnp.float32),
    )(x, partials, Ws, bs.reshape(1, D), Wn, bn.reshape(1, D))


def kernel(x, edge_index, edge_attr, W1, b1, W2, b2, Ws, bs, Wn, bn):
    w = _edge_mlp(edge_attr, W1, b1, W2, b2)
    partials = _sc_gather_scatter(edge_index[0], edge_index[1], x, w)
    return _out_lin(x, partials.reshape(NC, NP, D), Ws, bs, Wn, bn)
